# baseline (speedup 1.0000x reference)
import functools

import jax
import jax.numpy as jnp
from jax import lax
from jax.experimental import pallas as pl
from jax.experimental.pallas import tpu as pltpu
from jax.experimental.pallas import tpu_sc as plsc

D_MODEL = 64
NUM_CORES = 2
NUM_SUBCORES = 16
NUM_WORKERS = NUM_CORES * NUM_SUBCORES
IDXW = 64
WCHUNK = 320
NSTREAM = WCHUNK // IDXW
WIDEN_BB = 4000
REPACK_BB = 64


def _widen(table):
    vocab = table.shape[0]

    def body(i_ref, o_ref):
        x = i_ref[...]
        o_ref[...] = jnp.concatenate([x, x], axis=1)

    return pl.pallas_call(
        body,
        grid=(vocab // WIDEN_BB,),
        in_specs=[pl.BlockSpec((WIDEN_BB, D_MODEL), lambda i: (i, 0))],
        out_specs=pl.BlockSpec((WIDEN_BB, 2 * D_MODEL), lambda i: (i, 0)),
        out_shape=jax.ShapeDtypeStruct((vocab, 2 * D_MODEL), table.dtype),
    )(table)


def _gather(idx, table128, n):
    b_per_w = n // NUM_WORKERS
    nchunk = b_per_w // WCHUNK

    mesh = plsc.VectorSubcoreMesh(core_axis_name="c", subcore_axis_name="s")

    @functools.partial(
        pl.kernel,
        mesh=mesh,
        out_type=jax.ShapeDtypeStruct((n, 2 * D_MODEL), table128.dtype),
        scratch_types=[
            pltpu.VMEM((b_per_w,), jnp.int32),
            pltpu.VMEM((2, WCHUNK, 2 * D_MODEL), table128.dtype),
            pltpu.SemaphoreType.DMA((2,)),
            pltpu.SemaphoreType.DMA((2,)),
        ],
        compiler_params=pltpu.CompilerParams(use_tc_tiling_on_sc=False),
    )
    def gather_kernel(table_hbm, idx_hbm, out_hbm, idx_v, rows_v, gsem, wsem):
        wid = lax.axis_index("s") * NUM_CORES + lax.axis_index("c")
        base = wid * b_per_w
        pltpu.sync_copy(idx_hbm.at[pl.ds(base, b_per_w)], idx_v)

        def g_copy(c, slot, j):
            return pltpu.make_async_copy(
                table_hbm.at[idx_v.at[pl.ds(c * WCHUNK + j * IDXW, IDXW)]],
                rows_v.at[slot, pl.ds(j * IDXW, IDXW)],
                gsem.at[slot],
            )

        def startg(c, slot):
            for j in range(NSTREAM):
                g_copy(c, slot, j).start()

        def waitg(c, slot):
            for j in range(NSTREAM):
                g_copy(c, slot, j).wait()

        def w_copy(c, slot):
            return pltpu.make_async_copy(
                rows_v.at[slot],
                out_hbm.at[pl.ds(base + c * WCHUNK, WCHUNK)],
                wsem.at[slot],
            )

        startg(0, 0)

        @pl.loop(0, nchunk, step=2)
        def _(k):
            waitg(k, 0)
            w_copy(k, 0).start()

            @pl.when(k > 0)
            def _():
                w_copy(k - 1, 1).wait()

            startg(k + 1, 1)
            waitg(k + 1, 1)
            w_copy(k + 1, 1).start()
            w_copy(k, 0).wait()

            @pl.when(k + 2 < nchunk)
            def _():
                startg(k + 2, 0)

        w_copy(nchunk - 1, 1).wait()

    return gather_kernel(table128, idx)


def _repack(flat2d, batch, seq):
    rows_per_bb = REPACK_BB * seq

    def body(i_ref, o_ref):
        o_ref[...] = i_ref[:, :D_MODEL].reshape(o_ref.shape)

    return pl.pallas_call(
        body,
        grid=(batch // REPACK_BB,),
        in_specs=[pl.BlockSpec((rows_per_bb, 128), lambda i: (i, 0))],
        out_specs=pl.BlockSpec((REPACK_BB, seq, D_MODEL), lambda i: (i, 0, 0)),
        out_shape=jax.ShapeDtypeStruct((batch, seq, D_MODEL), flat2d.dtype),
    )(flat2d)


def kernel(x, table):
    batch, seq = x.shape
    n = batch * seq
    idx = x.reshape(n)
    table128 = _widen(table)
    flat2d = _gather(idx, table128, n)
    return _repack(flat2d, batch, seq)


# speedup vs baseline: 1.9842x; 1.9842x over previous
import functools

import jax
import jax.numpy as jnp
from jax import lax
from jax.experimental import pallas as pl
from jax.experimental.pallas import tpu as pltpu
from jax.experimental.pallas import tpu_sc as plsc

D_MODEL = 64
NUM_CORES = 2
NUM_SUBCORES = 16
NUM_WORKERS = NUM_CORES * NUM_SUBCORES
WCHUNK = 320
STREAM_W = (128, 128, 64)
STREAM_OFF = (0, 128, 256)
HALF = 1 << 19
WIDEN_BB = 16384
REPACK_BB = 64


def _widen(tableT):
    vocab = tableT.shape[1]
    nb_half = HALF // WIDEN_BB
    last_b = pl.cdiv(vocab, WIDEN_BB) - 1

    def body(lo_ref, hi_ref, o_ref):
        o_ref[...] = jnp.concatenate([lo_ref[...].T, hi_ref[...].T], axis=1)

    return pl.pallas_call(
        body,
        grid=(nb_half,),
        in_specs=[
            pl.BlockSpec((D_MODEL, WIDEN_BB), lambda i: (0, i)),
            pl.BlockSpec(
                (D_MODEL, WIDEN_BB),
                lambda i: (0, jnp.minimum(i + nb_half, last_b)),
            ),
        ],
        out_specs=pl.BlockSpec((WIDEN_BB, 2 * D_MODEL), lambda i: (i, 0)),
        out_shape=jax.ShapeDtypeStruct((HALF, 2 * D_MODEL), tableT.dtype),
    )(tableT, tableT)


def _gather(idx, table128, n):
    b_per_w = n // NUM_WORKERS
    nchunk = b_per_w // WCHUNK

    mesh = plsc.VectorSubcoreMesh(core_axis_name="c", subcore_axis_name="s")

    @functools.partial(
        pl.kernel,
        mesh=mesh,
        out_type=jax.ShapeDtypeStruct((n, 2 * D_MODEL), table128.dtype),
        scratch_types=[
            pltpu.VMEM((b_per_w,), jnp.int32),
            pltpu.VMEM((2, WCHUNK, 2 * D_MODEL), table128.dtype),
            pltpu.SemaphoreType.DMA((2,)),
            pltpu.SemaphoreType.DMA((2,)),
        ],
        compiler_params=pltpu.CompilerParams(use_tc_tiling_on_sc=False),
    )
    def gather_kernel(table_hbm, idx_hbm, out_hbm, idx_v, rows_v, gsem, wsem):
        wid = lax.axis_index("s") * NUM_CORES + lax.axis_index("c")
        base = wid * b_per_w
        pltpu.sync_copy(idx_hbm.at[pl.ds(base, b_per_w)], idx_v)

        def g_copy(c, slot, j):
            return pltpu.make_async_copy(
                table_hbm.at[idx_v.at[pl.ds(c * WCHUNK + STREAM_OFF[j], STREAM_W[j])]],
                rows_v.at[slot, pl.ds(STREAM_OFF[j], STREAM_W[j])],
                gsem.at[slot],
            )

        def startg(c, slot):
            for j in range(len(STREAM_W)):
                g_copy(c, slot, j).start()

        def waitg(c, slot):
            for j in range(len(STREAM_W)):
                g_copy(c, slot, j).wait()

        def w_copy(c, slot):
            return pltpu.make_async_copy(
                rows_v.at[slot],
                out_hbm.at[pl.ds(base + c * WCHUNK, WCHUNK)],
                wsem.at[slot],
            )

        startg(0, 0)

        @pl.loop(0, nchunk, step=2)
        def _(k):
            waitg(k, 0)
            w_copy(k, 0).start()

            @pl.when(k > 0)
            def _():
                w_copy(k - 1, 1).wait()

            startg(k + 1, 1)
            waitg(k + 1, 1)
            w_copy(k + 1, 1).start()
            w_copy(k, 0).wait()

            @pl.when(k + 2 < nchunk)
            def _():
                startg(k + 2, 0)

        w_copy(nchunk - 1, 1).wait()

    return gather_kernel(table128, idx)


def _repack(flat2d, sel, batch, seq):
    rows_per_bb = REPACK_BB * seq

    def body(i_ref, s_ref, o_ref):
        rows = i_ref[...].reshape(REPACK_BB, seq, 2 * D_MODEL)
        hi = s_ref[...][:, :, None] == 1
        o_ref[...] = jnp.where(hi, rows[..., D_MODEL:], rows[..., :D_MODEL])

    return pl.pallas_call(
        body,
        grid=(batch // REPACK_BB,),
        in_specs=[
            pl.BlockSpec((rows_per_bb, 128), lambda i: (i, 0)),
            pl.BlockSpec((REPACK_BB, seq), lambda i: (i, 0)),
        ],
        out_specs=pl.BlockSpec((REPACK_BB, seq, D_MODEL), lambda i: (i, 0, 0)),
        out_shape=jax.ShapeDtypeStruct((batch, seq, D_MODEL), flat2d.dtype),
    )(flat2d, sel)


def kernel(x, table):
    batch, seq = x.shape
    n = batch * seq
    idx = x.reshape(n)
    table128 = _widen(table.T)
    flat2d = _gather(idx & (HALF - 1), table128, n)
    return _repack(flat2d, x >> 19, batch, seq)


# speedup vs baseline: 2.0246x; 1.0204x over previous
import functools

import jax
import jax.numpy as jnp
from jax import lax
from jax.experimental import pallas as pl
from jax.experimental.pallas import tpu as pltpu
from jax.experimental.pallas import tpu_sc as plsc

D_MODEL = 64
NUM_CORES = 2
NUM_SUBCORES = 16
NUM_WORKERS = NUM_CORES * NUM_SUBCORES
WCHUNK = 320
STREAM_W = (128, 128, 64)
STREAM_OFF = (0, 128, 256)
NSLAB = 2
HALF = 1 << 19
WIDEN_BB = 16384
REPACK_BB = 64


def _widen(tableT):
    vocab = tableT.shape[1]
    nb_half = HALF // WIDEN_BB
    last_b = pl.cdiv(vocab, WIDEN_BB) - 1

    def body(lo_ref, hi_ref, o_ref):
        o_ref[...] = jnp.concatenate([lo_ref[...].T, hi_ref[...].T], axis=1)

    return pl.pallas_call(
        body,
        grid=(nb_half,),
        in_specs=[
            pl.BlockSpec((D_MODEL, WIDEN_BB), lambda i: (0, i)),
            pl.BlockSpec(
                (D_MODEL, WIDEN_BB),
                lambda i: (0, jnp.minimum(i + nb_half, last_b)),
            ),
        ],
        out_specs=pl.BlockSpec((WIDEN_BB, 2 * D_MODEL), lambda i: (i, 0)),
        out_shape=jax.ShapeDtypeStruct((HALF, 2 * D_MODEL), tableT.dtype),
    )(tableT, tableT)


def _gather(idx, table128, n):
    b_per_w = n // NUM_WORKERS
    nchunk = b_per_w // WCHUNK

    mesh = plsc.VectorSubcoreMesh(core_axis_name="c", subcore_axis_name="s")

    @functools.partial(
        pl.kernel,
        mesh=mesh,
        out_type=jax.ShapeDtypeStruct((n, 2 * D_MODEL), table128.dtype),
        scratch_types=[
            pltpu.VMEM((b_per_w,), jnp.int32),
            pltpu.VMEM((2, WCHUNK, 2 * D_MODEL), table128.dtype),
            pltpu.SemaphoreType.DMA((2,)),
            pltpu.SemaphoreType.DMA((2,)),
        ],
        compiler_params=pltpu.CompilerParams(use_tc_tiling_on_sc=False),
    )
    def gather_kernel(table_hbm, idx_hbm, out_hbm, idx_v, rows_v, gsem, wsem):
        wid = lax.axis_index("s") * NUM_CORES + lax.axis_index("c")
        base = wid * b_per_w
        pltpu.sync_copy(idx_hbm.at[pl.ds(base, b_per_w)], idx_v)

        def g_copy(c, slot, j):
            return pltpu.make_async_copy(
                table_hbm.at[idx_v.at[pl.ds(c * WCHUNK + STREAM_OFF[j], STREAM_W[j])]],
                rows_v.at[slot, pl.ds(STREAM_OFF[j], STREAM_W[j])],
                gsem.at[slot],
            )

        def startg(c, slot):
            for j in range(len(STREAM_W)):
                g_copy(c, slot, j).start()

        def waitg(c, slot):
            for j in range(len(STREAM_W)):
                g_copy(c, slot, j).wait()

        def w_copy(c, slot):
            return pltpu.make_async_copy(
                rows_v.at[slot],
                out_hbm.at[pl.ds(base + c * WCHUNK, WCHUNK)],
                wsem.at[slot],
            )

        startg(0, 0)

        @pl.loop(0, nchunk, step=2)
        def _(k):
            waitg(k, 0)
            w_copy(k, 0).start()

            @pl.when(k > 0)
            def _():
                w_copy(k - 1, 1).wait()

            startg(k + 1, 1)
            waitg(k + 1, 1)
            w_copy(k + 1, 1).start()
            w_copy(k, 0).wait()

            @pl.when(k + 2 < nchunk)
            def _():
                startg(k + 2, 0)

        w_copy(nchunk - 1, 1).wait()

    return gather_kernel(table128, idx)


def _repack(flat2d, sel, acc, slab, batch, seq):
    bslab = batch // NSLAB
    rows_per_bb = REPACK_BB * seq
    bb0 = slab * (bslab // REPACK_BB)

    def body(i_ref, s_ref, *rest):
        o_ref = rest[-1]
        rows = i_ref[...].reshape(REPACK_BB, seq, 2 * D_MODEL)
        hi = s_ref[...][:, :, None] == 1
        o_ref[...] = jnp.where(hi, rows[..., D_MODEL:], rows[..., :D_MODEL])

    in_specs = [
        pl.BlockSpec((rows_per_bb, 128), lambda i: (i, 0)),
        pl.BlockSpec((REPACK_BB, seq), lambda i: (i, 0)),
    ]
    operands = [flat2d, sel]
    aliases = {}
    if acc is not None:
        in_specs.append(pl.BlockSpec(memory_space=pl.ANY))
        operands.append(acc)
        aliases = {2: 0}

    return pl.pallas_call(
        body,
        grid=(bslab // REPACK_BB,),
        in_specs=in_specs,
        out_specs=pl.BlockSpec(
            (REPACK_BB, seq, D_MODEL), lambda i: (i + bb0, 0, 0)
        ),
        out_shape=jax.ShapeDtypeStruct((batch, seq, D_MODEL), flat2d.dtype),
        input_output_aliases=aliases,
    )(*operands)


def kernel(x, table):
    batch, seq = x.shape
    n = batch * seq
    nslab = n // NSLAB
    idx = x.reshape(n) & (HALF - 1)
    sel = x >> 19
    bslab = batch // NSLAB
    table128 = _widen(table.T)
    out = None
    for k in range(NSLAB):
        flat2d = _gather(idx[k * nslab:(k + 1) * nslab], table128, nslab)
        out = _repack(
            flat2d, sel[k * bslab:(k + 1) * bslab], out, k, batch, seq
        )
    return out


# speedup vs baseline: 2.0432x; 1.0092x over previous
import functools

import jax
import jax.numpy as jnp
from jax import lax
from jax.experimental import pallas as pl
from jax.experimental.pallas import tpu as pltpu
from jax.experimental.pallas import tpu_sc as plsc

D_MODEL = 64
NUM_CORES = 2
NUM_SUBCORES = 16
NUM_WORKERS = NUM_CORES * NUM_SUBCORES
WCHUNK = 200
STREAM_W = (128, 72)
STREAM_OFF = (0, 128)
NSLAB = 4
HALF = 1 << 19
WIDEN_BB = 16384
REPACK_BB = 64


def _widen(tableT):
    vocab = tableT.shape[1]
    nb_half = HALF // WIDEN_BB
    last_b = pl.cdiv(vocab, WIDEN_BB) - 1

    def body(lo_ref, hi_ref, o_ref):
        o_ref[...] = jnp.concatenate([lo_ref[...].T, hi_ref[...].T], axis=1)

    return pl.pallas_call(
        body,
        grid=(nb_half,),
        in_specs=[
            pl.BlockSpec((D_MODEL, WIDEN_BB), lambda i: (0, i)),
            pl.BlockSpec(
                (D_MODEL, WIDEN_BB),
                lambda i: (0, jnp.minimum(i + nb_half, last_b)),
            ),
        ],
        out_specs=pl.BlockSpec((WIDEN_BB, 2 * D_MODEL), lambda i: (i, 0)),
        out_shape=jax.ShapeDtypeStruct((HALF, 2 * D_MODEL), tableT.dtype),
    )(tableT, tableT)


def _gather(idx, table128, n):
    b_per_w = n // NUM_WORKERS
    nchunk = b_per_w // WCHUNK

    mesh = plsc.VectorSubcoreMesh(core_axis_name="c", subcore_axis_name="s")

    @functools.partial(
        pl.kernel,
        mesh=mesh,
        out_type=jax.ShapeDtypeStruct((n, 2 * D_MODEL), table128.dtype),
        scratch_types=[
            pltpu.VMEM((b_per_w,), jnp.int32),
            pltpu.VMEM((2, WCHUNK, 2 * D_MODEL), table128.dtype),
            pltpu.SemaphoreType.DMA((2,)),
            pltpu.SemaphoreType.DMA((2,)),
        ],
        compiler_params=pltpu.CompilerParams(use_tc_tiling_on_sc=False),
    )
    def gather_kernel(table_hbm, idx_hbm, out_hbm, idx_v, rows_v, gsem, wsem):
        wid = lax.axis_index("s") * NUM_CORES + lax.axis_index("c")
        base = wid * b_per_w
        pltpu.sync_copy(idx_hbm.at[pl.ds(base, b_per_w)], idx_v)

        def g_copy(c, slot, j):
            return pltpu.make_async_copy(
                table_hbm.at[idx_v.at[pl.ds(c * WCHUNK + STREAM_OFF[j], STREAM_W[j])]],
                rows_v.at[slot, pl.ds(STREAM_OFF[j], STREAM_W[j])],
                gsem.at[slot],
            )

        def startg(c, slot):
            for j in range(len(STREAM_W)):
                g_copy(c, slot, j).start()

        def waitg(c, slot):
            for j in range(len(STREAM_W)):
                g_copy(c, slot, j).wait()

        def w_copy(c, slot):
            return pltpu.make_async_copy(
                rows_v.at[slot],
                out_hbm.at[pl.ds(base + c * WCHUNK, WCHUNK)],
                wsem.at[slot],
            )

        startg(0, 0)

        @pl.loop(0, nchunk, step=2)
        def _(k):
            waitg(k, 0)
            w_copy(k, 0).start()

            @pl.when(k > 0)
            def _():
                w_copy(k - 1, 1).wait()

            startg(k + 1, 1)
            waitg(k + 1, 1)
            w_copy(k + 1, 1).start()
            w_copy(k, 0).wait()

            @pl.when(k + 2 < nchunk)
            def _():
                startg(k + 2, 0)

        w_copy(nchunk - 1, 1).wait()

    return gather_kernel(table128, idx)


def _repack(flat2d, sel, acc, slab, batch, seq):
    bslab = batch // NSLAB
    rows_per_bb = REPACK_BB * seq
    bb0 = slab * (bslab // REPACK_BB)

    def body(i_ref, s_ref, *rest):
        o_ref = rest[-1]
        rows = i_ref[...].reshape(REPACK_BB, seq, 2 * D_MODEL)
        hi = s_ref[...][:, :, None] == 1
        o_ref[...] = jnp.where(hi, rows[..., D_MODEL:], rows[..., :D_MODEL])

    in_specs = [
        pl.BlockSpec((rows_per_bb, 128), lambda i: (i, 0)),
        pl.BlockSpec((REPACK_BB, seq), lambda i: (i, 0)),
    ]
    operands = [flat2d, sel]
    aliases = {}
    if acc is not None:
        in_specs.append(pl.BlockSpec(memory_space=pl.ANY))
        operands.append(acc)
        aliases = {2: 0}

    return pl.pallas_call(
        body,
        grid=(bslab // REPACK_BB,),
        in_specs=in_specs,
        out_specs=pl.BlockSpec(
            (REPACK_BB, seq, D_MODEL), lambda i: (i + bb0, 0, 0)
        ),
        out_shape=jax.ShapeDtypeStruct((batch, seq, D_MODEL), flat2d.dtype),
        input_output_aliases=aliases,
    )(*operands)


def kernel(x, table):
    batch, seq = x.shape
    n = batch * seq
    nslab = n // NSLAB
    idx = x.reshape(n) & (HALF - 1)
    sel = x >> 19
    bslab = batch // NSLAB
    table128 = _widen(table.T)
    out = None
    for k in range(NSLAB):
        flat2d = _gather(idx[k * nslab:(k + 1) * nslab], table128, nslab)
        out = _repack(
            flat2d, sel[k * bslab:(k + 1) * bslab], out, k, batch, seq
        )
    return out
